# baseline (speedup 1.0000x reference)
import jax
import jax.numpy as jnp
from jax import lax
from jax.experimental import pallas as pl
from jax.experimental.pallas import tpu as pltpu
from jax.experimental.pallas import tpu_sc as plsc

RANK = 32
BATCH = 16384
NC = 2
NS = 16
NW = NC * NS
B_PER_W = BATCH // NW
N_VREG = B_PER_W // 16
IDX_CHUNK = 128
G_CHUNKS = (B_PER_W * RANK) // IDX_CHUNK
B_CHUNKS = B_PER_W // IDX_CHUNK
A_STRIDE = 7813 * 1024
S_STRIDE = 128


def _factorizer_kernel(ii_hbm, jj_hbm, ie_hbm, je_hbm, ib_hbm, jb_hbm,
                       const_hbm, out_hbm,
                       idxi_v, idxj_v, gi_v, gj_v, gdi_v, gdj_v,
                       ib_v, jb_v, const_v, out_v, sem):
  wid = lax.axis_index("s") * NC + lax.axis_index("c")
  base = wid * B_PER_W

  pltpu.sync_copy(ii_hbm.at[pl.ds(base, B_PER_W)], idxi_v)
  pltpu.sync_copy(jj_hbm.at[pl.ds(base, B_PER_W)], idxj_v)
  pltpu.sync_copy(const_hbm, const_v)

  for q in range(B_CHUNKS):
    sl = pl.ds(q * IDX_CHUNK, IDX_CHUNK)
    pltpu.async_copy(ib_hbm.at[idxi_v.at[sl]], ib_v.at[sl], sem)
    pltpu.async_copy(jb_hbm.at[idxj_v.at[sl]], jb_v.at[sl], sem)

  def build_body(v, carry):
    b0 = v * 16
    vi = idxi_v[pl.ds(b0, 16)]
    vj = idxj_v[pl.ds(b0, 16)]
    ti = vi
    tj = vj
    for a in range(4):
      for s in range(8):
        k = a * 8 + s
        off = k * 1000000
        gi_v[pl.ds(k * B_PER_W + b0, 16)] = ti + off
        gj_v[pl.ds(k * B_PER_W + b0, 16)] = tj + off
    return carry

  lax.fori_loop(0, N_VREG, build_body, 0)

  ie_flat = ie_hbm.at[0, pl.ds(0, 1000000)]
  je_flat = je_hbm.at[0, pl.ds(0, 1000000)]

  def fire_body(q, carry):
    sl = pl.ds(q * IDX_CHUNK, IDX_CHUNK)
    pltpu.async_copy(ie_flat.at[gi_v.at[sl]], gdi_v.at[sl], sem)
    pltpu.async_copy(je_flat.at[gj_v.at[sl]], gdj_v.at[sl], sem)
    return carry

  lax.fori_loop(0, G_CHUNKS, fire_body, 0)

  drain = pltpu.make_async_copy(
      ib_hbm.at[idxi_v.at[pl.ds(0, IDX_CHUNK)]],
      ib_v.at[pl.ds(0, IDX_CHUNK)], sem)

  def drain_body(q, carry):
    drain.wait()
    return carry

  lax.fori_loop(0, 2 * G_CHUNKS + 2 * B_CHUNKS, drain_body, 0)

  cvec = const_v[...]

  def group_body(g, carry):
    b0 = g * 16
    acc = jnp.zeros((16,), jnp.float32)
    for k in range(RANK):
      sl = pl.ds(k * B_PER_W + b0, 16)
      acc = acc + gdi_v[sl] * gdj_v[sl]
    out_v[pl.ds(b0, 16)] = (acc + ib_v[pl.ds(b0, 16)] + jb_v[pl.ds(b0, 16)]
                            + cvec)
    return carry

  lax.fori_loop(0, N_VREG, group_body, 0)

  pltpu.sync_copy(out_v, out_hbm.at[pl.ds(base, B_PER_W)])


@jax.jit
def _run(ii, jj, ie_t, je_t, i_bias_flat, j_bias_flat, const16):
  mesh = plsc.VectorSubcoreMesh(core_axis_name="c", subcore_axis_name="s")
  fn = pl.kernel(
      _factorizer_kernel,
      out_type=jax.ShapeDtypeStruct((BATCH,), jnp.float32),
      mesh=mesh,
      compiler_params=pltpu.CompilerParams(
          needs_layout_passes=False, use_tc_tiling_on_sc=False,
          disable_bounds_checks=True),
      scratch_types=[
          pltpu.VMEM((B_PER_W,), jnp.int32),
          pltpu.VMEM((B_PER_W,), jnp.int32),
          pltpu.VMEM((B_PER_W * RANK,), jnp.int32),
          pltpu.VMEM((B_PER_W * RANK,), jnp.int32),
          pltpu.VMEM((B_PER_W * RANK,), jnp.float32),
          pltpu.VMEM((B_PER_W * RANK,), jnp.float32),
          pltpu.VMEM((B_PER_W,), jnp.float32),
          pltpu.VMEM((B_PER_W,), jnp.float32),
          pltpu.VMEM((16,), jnp.float32),
          pltpu.VMEM((B_PER_W,), jnp.float32),
          pltpu.SemaphoreType.DMA,
      ],
  )
  return fn(ii, jj, ie_t, je_t, i_bias_flat, j_bias_flat, const16)


def kernel(inputs, i_embedding, j_embedding, i_bias, j_bias, constant):
  inputs = inputs.astype(jnp.int32)
  out = _run(inputs[:, 0], inputs[:, 1],
             jnp.swapaxes(i_embedding, 0, 1), jnp.swapaxes(j_embedding, 0, 1),
             i_bias.reshape(-1), j_bias.reshape(-1),
             jnp.broadcast_to(constant.reshape(-1), (16,)))
  return out.reshape(BATCH, 1)


# speedup vs baseline: 5.6956x; 5.6956x over previous
import jax
import jax.numpy as jnp
from jax import lax
from jax.experimental import pallas as pl
from jax.experimental.pallas import tpu as pltpu
from jax.experimental.pallas import tpu_sc as plsc

RANK = 32
BATCH = 16384
NC = 2
NS = 16
NW = NC * NS
B_PER_W = BATCH // NW
HALF = B_PER_W // 2
IDX_CHUNK = 128
B_CHUNKS = B_PER_W // IDX_CHUNK
ROW_W = 128


def _factorizer_kernel(ii_hbm, jj_hbm, ie4_hbm, je4_hbm, ib_hbm, jb_hbm,
                       const_hbm, out_hbm,
                       idxi_v, idxj_v, rowi_v, rowj_v, di_v, dj_v,
                       ib_v, jb_v, const_v, out_v, sem, sem2):
  wid = lax.axis_index("s") * NC + lax.axis_index("c")
  base = wid * B_PER_W

  pltpu.sync_copy(ii_hbm.at[pl.ds(base, B_PER_W)], idxi_v)
  pltpu.sync_copy(jj_hbm.at[pl.ds(base, B_PER_W)], idxj_v)
  pltpu.sync_copy(const_hbm, const_v)

  for q in range(B_CHUNKS):
    sl = pl.ds(q * IDX_CHUNK, IDX_CHUNK)
    pltpu.async_copy(ib_hbm.at[idxi_v.at[sl]], ib_v.at[sl], sem2)
    pltpu.async_copy(jb_hbm.at[idxj_v.at[sl]], jb_v.at[sl], sem2)

  def rows_body(v, carry):
    sl = pl.ds(v * 16, 16)
    rowi_v[sl] = idxi_v[sl] >> 2
    rowj_v[sl] = idxj_v[sl] >> 2
    return carry

  lax.fori_loop(0, B_PER_W // 16, rows_body, 0)

  cvec = const_v[...]
  lane = lax.iota(jnp.int32, 16)
  drain = pltpu.make_async_copy(
      ie4_hbm.at[rowi_v.at[pl.ds(0, IDX_CHUNK)]],
      di_v.at[pl.ds(0, IDX_CHUNK)], sem)
  drain_b = pltpu.make_async_copy(
      ib_hbm.at[idxi_v.at[pl.ds(0, IDX_CHUNK)]],
      ib_v.at[pl.ds(0, IDX_CHUNK)], sem2)

  for h in range(2):
    hb = h * HALF
    for q in range(HALF // IDX_CHUNK):
      isl = pl.ds(hb + q * IDX_CHUNK, IDX_CHUNK)
      dsl = pl.ds(q * IDX_CHUNK, IDX_CHUNK)
      pltpu.async_copy(ie4_hbm.at[rowi_v.at[isl]], di_v.at[dsl], sem)
      pltpu.async_copy(je4_hbm.at[rowj_v.at[isl]], dj_v.at[dsl], sem)
    for _ in range(2 * (HALF // IDX_CHUNK)):
      drain.wait()
    if h == 0:
      for _ in range(2 * B_CHUNKS):
        drain_b.wait()

    def group_body(g, carry):
      b0 = hb + g * 16
      vi = idxi_v[pl.ds(b0, 16)]
      vj = idxj_v[pl.ds(b0, 16)]
      rloc = lane + g * 16
      offi = (vi & 3) * 32
      offj = (vj & 3) * 32
      acc = jnp.zeros((16,), jnp.float32)
      for k in range(RANK):
        a = plsc.load_gather(di_v, [rloc, offi + k])
        b = plsc.load_gather(dj_v, [rloc, offj + k])
        acc = acc + a * b
      out_v[pl.ds(b0, 16)] = (acc + ib_v[pl.ds(b0, 16)] + jb_v[pl.ds(b0, 16)]
                              + cvec)
      return carry

    lax.fori_loop(0, HALF // 16, group_body, 0)

  pltpu.sync_copy(out_v, out_hbm.at[pl.ds(base, B_PER_W)])


@jax.jit
def _run(ii, jj, ie4, je4, i_bias_flat, j_bias_flat, const16):
  mesh = plsc.VectorSubcoreMesh(core_axis_name="c", subcore_axis_name="s")
  fn = pl.kernel(
      _factorizer_kernel,
      out_type=jax.ShapeDtypeStruct((BATCH,), jnp.float32),
      mesh=mesh,
      compiler_params=pltpu.CompilerParams(
          needs_layout_passes=False, use_tc_tiling_on_sc=False,
          disable_bounds_checks=True),
      scratch_types=[
          pltpu.VMEM((B_PER_W,), jnp.int32),
          pltpu.VMEM((B_PER_W,), jnp.int32),
          pltpu.VMEM((B_PER_W,), jnp.int32),
          pltpu.VMEM((B_PER_W,), jnp.int32),
          pltpu.VMEM((HALF, ROW_W), jnp.float32),
          pltpu.VMEM((HALF, ROW_W), jnp.float32),
          pltpu.VMEM((B_PER_W,), jnp.float32),
          pltpu.VMEM((B_PER_W,), jnp.float32),
          pltpu.VMEM((16,), jnp.float32),
          pltpu.VMEM((B_PER_W,), jnp.float32),
          pltpu.SemaphoreType.DMA,
          pltpu.SemaphoreType.DMA,
      ],
  )
  return fn(ii, jj, ie4, je4, i_bias_flat, j_bias_flat, const16)


def kernel(inputs, i_embedding, j_embedding, i_bias, j_bias, constant):
  inputs = inputs.astype(jnp.int32)
  out = _run(inputs[:, 0], inputs[:, 1],
             i_embedding.reshape(250000, 128), j_embedding.reshape(250000, 128),
             i_bias.reshape(-1), j_bias.reshape(-1),
             jnp.broadcast_to(constant.reshape(-1), (16,)))
  return out.reshape(BATCH, 1)


# speedup vs baseline: 6.7640x; 1.1876x over previous
import jax
import jax.numpy as jnp
from jax import lax
from jax.experimental import pallas as pl
from jax.experimental.pallas import tpu as pltpu
from jax.experimental.pallas import tpu_sc as plsc

RANK = 32
BATCH = 16384
DIM = 1000000
NC = 2
NS = 16
NW = NC * NS
B_PER_W = BATCH // NW
N_VREG = B_PER_W // 16
IDX_CHUNK = 128
G_CHUNKS = (B_PER_W * RANK) // IDX_CHUNK
B_CHUNKS = B_PER_W // IDX_CHUNK

CBLK = 64
NCB = 123
BAND_ROWS = NCB * CBLK * 8
A_STRIDE = BAND_ROWS * 128
S_STRIDE = 128
FLAT = 4 * A_STRIDE


def _detile_kernel(x_ref, o_ref):
  x = x_ref[...]
  o_ref[...] = jnp.swapaxes(
      x.reshape(8, CBLK, 128), 0, 1).reshape(1, CBLK * 8, 128)


def _detile(table_t):
  return pl.pallas_call(
      _detile_kernel,
      grid=(4, NCB),
      in_specs=[pl.BlockSpec((8, CBLK * 128), lambda a, cb: (a, cb))],
      out_specs=pl.BlockSpec((1, CBLK * 8, 128), lambda a, cb: (a, cb, 0)),
      out_shape=jax.ShapeDtypeStruct((4, BAND_ROWS, 128), jnp.float32),
      compiler_params=pltpu.CompilerParams(
          dimension_semantics=("arbitrary", "arbitrary")),
  )(table_t)


def _factorizer_kernel(ii_hbm, jj_hbm, ie_hbm, je_hbm, ib_hbm, jb_hbm,
                       const_hbm, out_hbm,
                       idxi_v, idxj_v, gi_v, gj_v, gdi_v, gdj_v,
                       ib_v, jb_v, const_v, out_v, sem):
  wid = lax.axis_index("s") * NC + lax.axis_index("c")
  base = wid * B_PER_W

  pltpu.sync_copy(ii_hbm.at[pl.ds(base, B_PER_W)], idxi_v)
  pltpu.sync_copy(jj_hbm.at[pl.ds(base, B_PER_W)], idxj_v)
  pltpu.sync_copy(const_hbm, const_v)

  for q in range(B_CHUNKS):
    sl = pl.ds(q * IDX_CHUNK, IDX_CHUNK)
    pltpu.async_copy(ib_hbm.at[idxi_v.at[sl]], ib_v.at[sl], sem)
    pltpu.async_copy(jb_hbm.at[idxj_v.at[sl]], jb_v.at[sl], sem)

  def build_body(v, carry):
    b0 = v * 16
    vi = idxi_v[pl.ds(b0, 16)]
    vj = idxj_v[pl.ds(b0, 16)]
    ti = ((vi >> 7) << 10) + (vi & 127)
    tj = ((vj >> 7) << 10) + (vj & 127)
    for a in range(4):
      for s in range(8):
        off = a * A_STRIDE + s * S_STRIDE
        k = a * 8 + s
        gi_v[pl.ds(k * B_PER_W + b0, 16)] = ti + off
        gj_v[pl.ds(k * B_PER_W + b0, 16)] = tj + off
    return carry

  lax.fori_loop(0, N_VREG, build_body, 0)

  def fire_body(q, carry):
    sl = pl.ds(q * IDX_CHUNK, IDX_CHUNK)
    pltpu.async_copy(ie_hbm.at[gi_v.at[sl]], gdi_v.at[sl], sem)
    pltpu.async_copy(je_hbm.at[gj_v.at[sl]], gdj_v.at[sl], sem)
    return carry

  lax.fori_loop(0, G_CHUNKS, fire_body, 0)

  drain = pltpu.make_async_copy(
      ib_hbm.at[idxi_v.at[pl.ds(0, IDX_CHUNK)]],
      ib_v.at[pl.ds(0, IDX_CHUNK)], sem)

  def drain_body(q, carry):
    drain.wait()
    return carry

  lax.fori_loop(0, 2 * G_CHUNKS + 2 * B_CHUNKS, drain_body, 0)

  cvec = const_v[...]

  def group_body(g, carry):
    b0 = g * 16
    acc = jnp.zeros((16,), jnp.float32)
    for k in range(RANK):
      sl = pl.ds(k * B_PER_W + b0, 16)
      acc = acc + gdi_v[sl] * gdj_v[sl]
    out_v[pl.ds(b0, 16)] = (acc + ib_v[pl.ds(b0, 16)] + jb_v[pl.ds(b0, 16)]
                            + cvec)
    return carry

  lax.fori_loop(0, N_VREG, group_body, 0)

  pltpu.sync_copy(out_v, out_hbm.at[pl.ds(base, B_PER_W)])


def _gather_dot(ii, jj, ie_flat, je_flat, i_bias_flat, j_bias_flat, const16):
  mesh = plsc.VectorSubcoreMesh(core_axis_name="c", subcore_axis_name="s")
  fn = pl.kernel(
      _factorizer_kernel,
      out_type=jax.ShapeDtypeStruct((BATCH,), jnp.float32),
      mesh=mesh,
      compiler_params=pltpu.CompilerParams(
          needs_layout_passes=False, use_tc_tiling_on_sc=False,
          disable_bounds_checks=True),
      scratch_types=[
          pltpu.VMEM((B_PER_W,), jnp.int32),
          pltpu.VMEM((B_PER_W,), jnp.int32),
          pltpu.VMEM((B_PER_W * RANK,), jnp.int32),
          pltpu.VMEM((B_PER_W * RANK,), jnp.int32),
          pltpu.VMEM((B_PER_W * RANK,), jnp.float32),
          pltpu.VMEM((B_PER_W * RANK,), jnp.float32),
          pltpu.VMEM((B_PER_W,), jnp.float32),
          pltpu.VMEM((B_PER_W,), jnp.float32),
          pltpu.VMEM((16,), jnp.float32),
          pltpu.VMEM((B_PER_W,), jnp.float32),
          pltpu.SemaphoreType.DMA,
      ],
  )
  return fn(ii, jj, ie_flat, je_flat, i_bias_flat, j_bias_flat, const16)


@jax.jit
def _run(inputs, i_embedding, j_embedding, i_bias, j_bias, constant):
  ie_lin = _detile(jnp.swapaxes(i_embedding, 0, 1)).reshape(-1)
  je_lin = _detile(jnp.swapaxes(j_embedding, 0, 1)).reshape(-1)
  return _gather_dot(inputs[:, 0], inputs[:, 1], ie_lin, je_lin,
                     i_bias.reshape(-1), j_bias.reshape(-1),
                     jnp.broadcast_to(constant.reshape(-1), (16,)))


def kernel(inputs, i_embedding, j_embedding, i_bias, j_bias, constant):
  out = _run(inputs.astype(jnp.int32), i_embedding, j_embedding,
             i_bias, j_bias, constant)
  return out.reshape(BATCH, 1)


# speedup vs baseline: 16.8966x; 2.4980x over previous
import jax
import jax.numpy as jnp
from jax import lax
from jax.experimental import pallas as pl
from jax.experimental.pallas import tpu as pltpu
from jax.experimental.pallas import tpu_sc as plsc

RANK = 32
BATCH = 16384
NC = 2
NS = 16
NW = NC * NS
B_PER_W = BATCH // NW
W_PER_W = B_PER_W * RANK
N_VREG = B_PER_W // 16
IDX_CHUNK = 128
G_CHUNKS = W_PER_W // IDX_CHUNK
B_CHUNKS = B_PER_W // IDX_CHUNK

CBLK = 2048
NCB = 4
S_STRIDE = NCB * CBLK * 128
A_STRIDE = 8 * S_STRIDE


def _detile_kernel(x_ref, o_ref):
  o_ref[...] = x_ref[...].reshape(1, 8, CBLK, 128)


def _detile(table_t):
  return pl.pallas_call(
      _detile_kernel,
      grid=(4, NCB),
      in_specs=[pl.BlockSpec((8, CBLK * 128), lambda a, cb: (a, cb))],
      out_specs=pl.BlockSpec((1, 8, CBLK, 128), lambda a, cb: (a, 0, cb, 0)),
      out_shape=jax.ShapeDtypeStruct((4, 8, NCB * CBLK, 128), jnp.float32),
      compiler_params=pltpu.CompilerParams(
          dimension_semantics=("parallel", "parallel")),
  )(table_t)


def _build_offsets(idx_v, g_v):
  def build_body(v, carry):
    b0 = v * 16
    vi = idx_v[pl.ds(b0, 16)]
    for a in range(4):
      for s in range(8):
        k = a * 8 + s
        g_v[pl.ds(k * B_PER_W + b0, 16)] = vi + (a * A_STRIDE + s * S_STRIDE)
    return carry

  lax.fori_loop(0, N_VREG, build_body, 0)


def _stage_a_kernel(ii_hbm, ie_hbm, ib_hbm, gd_hbm, ibg_hbm,
                    idx_v, g_v, gd_v, ib_v, sem):
  wid = lax.axis_index("s") * NC + lax.axis_index("c")
  base = wid * B_PER_W

  pltpu.sync_copy(ii_hbm.at[pl.ds(base, B_PER_W)], idx_v)

  for q in range(B_CHUNKS):
    sl = pl.ds(q * IDX_CHUNK, IDX_CHUNK)
    pltpu.async_copy(ib_hbm.at[idx_v.at[sl]], ib_v.at[sl], sem)

  _build_offsets(idx_v, g_v)

  def fire_body(q, carry):
    sl = pl.ds(q * IDX_CHUNK, IDX_CHUNK)
    pltpu.async_copy(ie_hbm.at[g_v.at[sl]], gd_v.at[sl], sem)
    return carry

  lax.fori_loop(0, G_CHUNKS, fire_body, 0)

  drain = pltpu.make_async_copy(
      ib_hbm.at[idx_v.at[pl.ds(0, IDX_CHUNK)]],
      ib_v.at[pl.ds(0, IDX_CHUNK)], sem)

  def drain_body(q, carry):
    drain.wait()
    return carry

  lax.fori_loop(0, G_CHUNKS + B_CHUNKS, drain_body, 0)

  pltpu.sync_copy(gd_v, gd_hbm.at[pl.ds(wid * W_PER_W, W_PER_W)])
  pltpu.sync_copy(ib_v, ibg_hbm.at[pl.ds(base, B_PER_W)])


def _stage_b_kernel(jj_hbm, je_hbm, jb_hbm, gd_hbm, ibg_hbm, const_hbm,
                    out_hbm,
                    idx_v, g_v, gdj_v, gdi_v, ib_v, jb_v, const_v, out_v,
                    sem):
  wid = lax.axis_index("s") * NC + lax.axis_index("c")
  base = wid * B_PER_W

  pltpu.sync_copy(jj_hbm.at[pl.ds(base, B_PER_W)], idx_v)
  pltpu.sync_copy(gd_hbm.at[pl.ds(wid * W_PER_W, W_PER_W)], gdi_v)
  pltpu.sync_copy(ibg_hbm.at[pl.ds(base, B_PER_W)], ib_v)
  pltpu.sync_copy(const_hbm, const_v)

  for q in range(B_CHUNKS):
    sl = pl.ds(q * IDX_CHUNK, IDX_CHUNK)
    pltpu.async_copy(jb_hbm.at[idx_v.at[sl]], jb_v.at[sl], sem)

  _build_offsets(idx_v, g_v)

  def fire_body(q, carry):
    sl = pl.ds(q * IDX_CHUNK, IDX_CHUNK)
    pltpu.async_copy(je_hbm.at[g_v.at[sl]], gdj_v.at[sl], sem)
    return carry

  lax.fori_loop(0, G_CHUNKS, fire_body, 0)

  drain = pltpu.make_async_copy(
      jb_hbm.at[idx_v.at[pl.ds(0, IDX_CHUNK)]],
      jb_v.at[pl.ds(0, IDX_CHUNK)], sem)

  def drain_body(q, carry):
    drain.wait()
    return carry

  lax.fori_loop(0, G_CHUNKS + B_CHUNKS, drain_body, 0)

  cvec = const_v[...]

  def group_body(g, carry):
    b0 = g * 16
    acc = jnp.zeros((16,), jnp.float32)
    for k in range(RANK):
      sl = pl.ds(k * B_PER_W + b0, 16)
      acc = acc + gdi_v[sl] * gdj_v[sl]
    out_v[pl.ds(b0, 16)] = (acc + ib_v[pl.ds(b0, 16)] + jb_v[pl.ds(b0, 16)]
                            + cvec)
    return carry

  lax.fori_loop(0, N_VREG, group_body, 0)

  pltpu.sync_copy(out_v, out_hbm.at[pl.ds(base, B_PER_W)])


_SC_COMPILER_PARAMS = pltpu.CompilerParams(
    needs_layout_passes=False, use_tc_tiling_on_sc=False,
    disable_bounds_checks=True)


def _stage_a(ii, ie_flat, i_bias_flat):
  mesh = plsc.VectorSubcoreMesh(core_axis_name="c", subcore_axis_name="s")
  fn = pl.kernel(
      _stage_a_kernel,
      out_type=(jax.ShapeDtypeStruct((BATCH * RANK,), jnp.float32),
                jax.ShapeDtypeStruct((BATCH,), jnp.float32)),
      mesh=mesh,
      compiler_params=_SC_COMPILER_PARAMS,
      scratch_types=[
          pltpu.VMEM((B_PER_W,), jnp.int32),
          pltpu.VMEM((W_PER_W,), jnp.int32),
          pltpu.VMEM((W_PER_W,), jnp.float32),
          pltpu.VMEM((B_PER_W,), jnp.float32),
          pltpu.SemaphoreType.DMA,
      ],
  )
  return fn(ii, ie_flat, i_bias_flat)


def _stage_b(jj, je_flat, j_bias_flat, gd, ibg, const16):
  mesh = plsc.VectorSubcoreMesh(core_axis_name="c", subcore_axis_name="s")
  fn = pl.kernel(
      _stage_b_kernel,
      out_type=jax.ShapeDtypeStruct((BATCH,), jnp.float32),
      mesh=mesh,
      compiler_params=_SC_COMPILER_PARAMS,
      scratch_types=[
          pltpu.VMEM((B_PER_W,), jnp.int32),
          pltpu.VMEM((W_PER_W,), jnp.int32),
          pltpu.VMEM((W_PER_W,), jnp.float32),
          pltpu.VMEM((W_PER_W,), jnp.float32),
          pltpu.VMEM((B_PER_W,), jnp.float32),
          pltpu.VMEM((B_PER_W,), jnp.float32),
          pltpu.VMEM((16,), jnp.float32),
          pltpu.VMEM((B_PER_W,), jnp.float32),
          pltpu.SemaphoreType.DMA,
      ],
  )
  return fn(jj, je_flat, j_bias_flat, gd, ibg, const16)


@jax.jit
def _run(inputs, i_embedding, j_embedding, i_bias, j_bias, constant):
  ie_lin = _detile(jnp.swapaxes(i_embedding, 0, 1)).reshape(-1)
  gd, ibg = _stage_a(inputs[:, 0], ie_lin, i_bias.reshape(-1))
  je_lin = _detile(jnp.swapaxes(j_embedding, 0, 1)).reshape(-1)
  return _stage_b(inputs[:, 1], je_lin, j_bias.reshape(-1), gd, ibg,
                  jnp.broadcast_to(constant.reshape(-1), (16,)))


def kernel(inputs, i_embedding, j_embedding, i_bias, j_bias, constant):
  out = _run(inputs.astype(jnp.int32), i_embedding, j_embedding,
             i_bias, j_bias, constant)
  return out.reshape(BATCH, 1)
